# baseline (speedup 1.0000x reference)
import functools

import jax
import jax.numpy as jnp
from jax import lax
from jax.experimental import pallas as pl
from jax.experimental.pallas import tpu as pltpu
from jax.experimental.pallas import tpu_sc as plsc

_B, _T, _D = 4, 8192, 768
_NC = 1000
_N = 4
_K = 16
_TBLK = 256
_NCHUNK = 8
_CHUNK = _T // _NCHUNK
_NEG = -3.0e38


_NBLK = _T // _TBLK
_NCACHE = 18


def _stats_body(e_ref, mu_ref, dev_ref, cache_ref):
    p = pl.program_id(0)
    i = pl.program_id(1)

    @pl.when(p == 0)
    def _():
        s = jnp.sum(e_ref[...], axis=1)

        @pl.when(i == 0)
        def _():
            mu_ref[...] = jnp.zeros_like(mu_ref)

        mu_ref[...] += s

        @pl.when(i < _NCACHE)
        def _():
            cache_ref[:, pl.ds(i * _TBLK, _TBLK), :] = e_ref[...]

        @pl.when(i == pl.num_programs(1) - 1)
        def _():
            mu_ref[...] *= jnp.float32(1.0 / _T)

    @pl.when(p == 1)
    def _():
        mu = mu_ref[...][:, None, :]

        @pl.when(i < _NBLK - _NCACHE)
        def _():
            d = e_ref[...] - mu
            dev_ref[...] = jnp.sum(d * d, axis=-1)

        @pl.when(i >= _NBLK - _NCACHE)
        def _():
            j = i - (_NBLK - _NCACHE)
            d = cache_ref[:, pl.ds(j * _TBLK, _TBLK), :] - mu
            dev_ref[...] = jnp.sum(d * d, axis=-1)


def _stats(E):
    def e_map(p, i):
        late = jnp.where(i < _NBLK - _NCACHE, i + _NCACHE, _NBLK - 1)
        return (0, jnp.where(p == 0, i, late), 0)

    def dev_map(p, i):
        ph1 = jnp.where(i < _NBLK - _NCACHE, i + _NCACHE, i - (_NBLK - _NCACHE))
        return (0, jnp.where(p == 0, 0, ph1))

    return pl.pallas_call(
        _stats_body,
        grid=(2, _NBLK),
        in_specs=[pl.BlockSpec((_B, _TBLK, _D), e_map)],
        out_specs=[
            pl.BlockSpec((_B, _D), lambda p, i: (0, 0)),
            pl.BlockSpec((_B, _TBLK), dev_map),
        ],
        out_shape=[
            jax.ShapeDtypeStruct((_B, _D), jnp.float32),
            jax.ShapeDtypeStruct((_B, _T), jnp.float32),
        ],
        scratch_shapes=[pltpu.VMEM((_B, _NCACHE * _TBLK, _D), jnp.float32)],
        compiler_params=pltpu.CompilerParams(
            vmem_limit_bytes=112 * 1024 * 1024),
    )(E)


def _merge16(ck, ci, nk, ni):
    kd, idd = plsc.sort_key_val(nk, ni, descending=True)
    ge = ck >= kd
    mk = jnp.where(ge, ck, kd)
    mi = jnp.where(ge, ci, idd)
    return tuple(plsc.sort_key_val(mk, mi))


def _sc_topk_body(dev_hbm, e_hbm, out_hbm, devv, rows, ssum, sem):
    c = lax.axis_index("c")
    s = lax.axis_index("s")
    sample = 2 * c + s // _NCHUNK

    @pl.when((s == 0) | (s == _NCHUNK))
    def _():
        pltpu.sync_copy(dev_hbm.at[pl.ds(sample * _T, _T)], devv)
        iota = lax.broadcasted_iota(jnp.int32, (16,), 0)

        def body(j, carry):
            ck, ci = carry
            v = devv[pl.ds(j * 16, 16)]
            return _merge16(ck, ci, v, iota + j * 16)

        init = (jnp.full((16,), _NEG, jnp.float32),
                jnp.zeros((16,), jnp.int32))
        _, fi = lax.fori_loop(0, _T // 16, body, init)

        pltpu.sync_copy(e_hbm.at[fi + sample * _T], rows)

        def dsum(dj, _):
            def inner(r, acc):
                return acc + rows[r, pl.ds(dj * 16, 16)]

            ssum[pl.ds(dj * 16, 16)] = lax.fori_loop(
                0, _K, inner, jnp.zeros((16,), jnp.float32))
            return 0

        lax.fori_loop(0, _D // 16, dsum, 0)
        pltpu.sync_copy(ssum, out_hbm.at[sample])


def _sc_topk(dev2_flat, e_flat):
    kern = pl.kernel(
        _sc_topk_body,
        out_type=jax.ShapeDtypeStruct((_B, _D), jnp.float32),
        mesh=plsc.VectorSubcoreMesh(core_axis_name="c", subcore_axis_name="s"),
        compiler_params=pltpu.CompilerParams(needs_layout_passes=False),
        scratch_types=[
            pltpu.VMEM((_T,), jnp.float32),
            pltpu.VMEM((_K, _D), jnp.float32),
            pltpu.VMEM((_D,), jnp.float32),
            pltpu.SemaphoreType.DMA,
        ],
    )
    return kern(dev2_flat, e_flat)


def _head_body(mu_ref, s_ref, w1_ref, b1_ref, w2_ref, b2_ref, out_ref):
    mu = mu_ref[...]
    S = s_ref[...]
    corr = (S - jnp.float32(_K) * mu) * jnp.float32(1.0 / _T)
    mu_rep = jnp.broadcast_to(mu[:, None, :], (_B, _N, _D)).reshape(_B * _N, _D)
    co_rep = jnp.broadcast_to(corr[:, None, :], (_B, _N, _D)).reshape(_B * _N, _D)
    ii = lax.broadcasted_iota(jnp.int32, (_B * _N, 1), 0)
    alpha = (ii % _N).astype(jnp.float32) * jnp.float32(1.0 / (_N - 1))
    Z = mu_rep - alpha * co_rep
    h = lax.dot_general(Z, w1_ref[...], (((1,), (1,)), ((), ())),
                        preferred_element_type=jnp.float32) + b1_ref[...]
    h = 0.5 * h * (1.0 + lax.erf(h * jnp.float32(0.7071067811865476)))
    out_ref[...] = lax.dot_general(h, w2_ref[...], (((1,), (1,)), ((), ())),
                                   preferred_element_type=jnp.float32) + b2_ref[...]


def _head(mu, S, W1, b1, W2, b2):
    return pl.pallas_call(
        _head_body,
        out_shape=jax.ShapeDtypeStruct((_B * _N, _NC), jnp.float32),
    )(mu, S, W1, b1.reshape(1, -1), W2, b2.reshape(1, -1))


def kernel(E, W1, b1, W2, b2):
    mu, dev2 = _stats(E)
    S = _sc_topk(dev2.reshape(-1), E.reshape(_B * _T, _D))
    return _head(mu, S, W1, b1, W2, b2)


# speedup vs baseline: 1.0636x; 1.0636x over previous
import functools

import jax
import jax.numpy as jnp
from jax import lax
from jax.experimental import pallas as pl
from jax.experimental.pallas import tpu as pltpu
from jax.experimental.pallas import tpu_sc as plsc

_B, _T, _D = 4, 8192, 768
_NC = 1000
_N = 4
_K = 16
_TBLK = 512
_NCHUNK = 8
_CHUNK = _T // _NCHUNK
_NEG = -3.0e38


_NBLK = _T // _TBLK
_NCACHE = 8


def _stats_body(e_ref, mu_ref, dev_ref, cache_ref):
    p = pl.program_id(0)
    i = pl.program_id(1)

    @pl.when(p == 0)
    def _():
        s = jnp.sum(e_ref[...], axis=1)

        @pl.when(i == 0)
        def _():
            mu_ref[...] = jnp.zeros_like(mu_ref)

        mu_ref[...] += s

        @pl.when(i < _NCACHE)
        def _():
            cache_ref[:, pl.ds(i * _TBLK, _TBLK), :] = e_ref[...]

        @pl.when(i == pl.num_programs(1) - 1)
        def _():
            mu_ref[...] *= jnp.float32(1.0 / _T)

    @pl.when(p == 1)
    def _():
        mu = mu_ref[...][:, None, :]

        @pl.when(i % 2 == 0)
        def _():
            d = e_ref[...] - mu
            dev_ref[...] = jnp.sum(d * d, axis=-1)

        @pl.when(i % 2 == 1)
        def _():
            j = i // 2
            d = cache_ref[:, pl.ds(j * _TBLK, _TBLK), :] - mu
            dev_ref[...] = jnp.sum(d * d, axis=-1)


def _stats(E):
    def e_map(p, i):
        return (0, jnp.where(p == 0, i, _NCACHE + i // 2), 0)

    def dev_map(p, i):
        ph1 = jnp.where(i % 2 == 0, _NCACHE + i // 2, i // 2)
        return (0, jnp.where(p == 0, 0, ph1))

    return pl.pallas_call(
        _stats_body,
        grid=(2, _NBLK),
        in_specs=[pl.BlockSpec((_B, _TBLK, _D), e_map)],
        out_specs=[
            pl.BlockSpec((_B, _D), lambda p, i: (0, 0)),
            pl.BlockSpec((_B, _TBLK), dev_map),
        ],
        out_shape=[
            jax.ShapeDtypeStruct((_B, _D), jnp.float32),
            jax.ShapeDtypeStruct((_B, _T), jnp.float32),
        ],
        scratch_shapes=[pltpu.VMEM((_B, _NCACHE * _TBLK, _D), jnp.float32)],
        compiler_params=pltpu.CompilerParams(
            vmem_limit_bytes=112 * 1024 * 1024),
    )(E)


def _merge16(ck, ci, nk, ni):
    kd, idd = plsc.sort_key_val(nk, ni, descending=True)
    ge = ck >= kd
    mk = jnp.where(ge, ck, kd)
    mi = jnp.where(ge, ci, idd)
    return tuple(plsc.sort_key_val(mk, mi))


def _sc_topk_body(dev_hbm, e_hbm, out_hbm, devv, rows, ssum, sem):
    c = lax.axis_index("c")
    s = lax.axis_index("s")
    sample = 2 * c + s // _NCHUNK

    @pl.when((s == 0) | (s == _NCHUNK))
    def _():
        pltpu.sync_copy(dev_hbm.at[pl.ds(sample * _T, _T)], devv)
        iota = lax.broadcasted_iota(jnp.int32, (16,), 0)

        def body(j, carry):
            ck, ci = carry
            v = devv[pl.ds(j * 16, 16)]
            return _merge16(ck, ci, v, iota + j * 16)

        init = (jnp.full((16,), _NEG, jnp.float32),
                jnp.zeros((16,), jnp.int32))
        _, fi = lax.fori_loop(0, _T // 16, body, init)

        pltpu.sync_copy(e_hbm.at[fi + sample * _T], rows)

        def dsum(dj, _):
            def inner(r, acc):
                return acc + rows[r, pl.ds(dj * 16, 16)]

            ssum[pl.ds(dj * 16, 16)] = lax.fori_loop(
                0, _K, inner, jnp.zeros((16,), jnp.float32))
            return 0

        lax.fori_loop(0, _D // 16, dsum, 0)
        pltpu.sync_copy(ssum, out_hbm.at[sample])


def _sc_topk(dev2_flat, e_flat):
    kern = pl.kernel(
        _sc_topk_body,
        out_type=jax.ShapeDtypeStruct((_B, _D), jnp.float32),
        mesh=plsc.VectorSubcoreMesh(core_axis_name="c", subcore_axis_name="s"),
        compiler_params=pltpu.CompilerParams(needs_layout_passes=False),
        scratch_types=[
            pltpu.VMEM((_T,), jnp.float32),
            pltpu.VMEM((_K, _D), jnp.float32),
            pltpu.VMEM((_D,), jnp.float32),
            pltpu.SemaphoreType.DMA,
        ],
    )
    return kern(dev2_flat, e_flat)


def _head_body(mu_ref, s_ref, w1_ref, b1_ref, w2_ref, b2_ref, out_ref):
    mu = mu_ref[...]
    S = s_ref[...]
    corr = (S - jnp.float32(_K) * mu) * jnp.float32(1.0 / _T)
    mu_rep = jnp.broadcast_to(mu[:, None, :], (_B, _N, _D)).reshape(_B * _N, _D)
    co_rep = jnp.broadcast_to(corr[:, None, :], (_B, _N, _D)).reshape(_B * _N, _D)
    ii = lax.broadcasted_iota(jnp.int32, (_B * _N, 1), 0)
    alpha = (ii % _N).astype(jnp.float32) * jnp.float32(1.0 / (_N - 1))
    Z = mu_rep - alpha * co_rep
    h = lax.dot_general(Z, w1_ref[...], (((1,), (1,)), ((), ())),
                        preferred_element_type=jnp.float32) + b1_ref[...]
    h = 0.5 * h * (1.0 + lax.erf(h * jnp.float32(0.7071067811865476)))
    out_ref[...] = lax.dot_general(h, w2_ref[...], (((1,), (1,)), ((), ())),
                                   preferred_element_type=jnp.float32) + b2_ref[...]


def _head(mu, S, W1, b1, W2, b2):
    return pl.pallas_call(
        _head_body,
        out_shape=jax.ShapeDtypeStruct((_B * _N, _NC), jnp.float32),
    )(mu, S, W1, b1.reshape(1, -1), W2, b2.reshape(1, -1))


def kernel(E, W1, b1, W2, b2):
    mu, dev2 = _stats(E)
    S = _sc_topk(dev2.reshape(-1), E.reshape(_B * _T, _D))
    return _head(mu, S, W1, b1, W2, b2)


# speedup vs baseline: 1.1386x; 1.0705x over previous
import functools

import jax
import jax.numpy as jnp
from jax import lax
from jax.experimental import pallas as pl
from jax.experimental.pallas import tpu as pltpu
from jax.experimental.pallas import tpu_sc as plsc

_B, _T, _D = 4, 8192, 768
_NC = 1000
_N = 4
_K = 16
_TBLK = 512
_NCHUNK = 8
_CHUNK = _T // _NCHUNK
_NEG = -3.0e38


_NBLK = _T // _TBLK
_NCACHE = 8


def _stats_body(e_ref, mu_ref, dev_ref, cache_ref):
    p = pl.program_id(0)
    i = pl.program_id(1)

    @pl.when(p == 0)
    def _():
        s = jnp.sum(e_ref[...], axis=1)

        @pl.when(i == 0)
        def _():
            mu_ref[...] = jnp.zeros_like(mu_ref)

        mu_ref[...] += s

        @pl.when(i < _NCACHE)
        def _():
            cache_ref[:, pl.ds(i * _TBLK, _TBLK), :] = e_ref[...]

        @pl.when(i == pl.num_programs(1) - 1)
        def _():
            mu_ref[...] *= jnp.float32(1.0 / _T)

    @pl.when(p == 1)
    def _():
        mu = mu_ref[...][:, None, :]

        @pl.when(i < _NBLK - _NCACHE)
        def _():
            d = e_ref[...] - mu
            dev_ref[...] = jnp.sum(d * d, axis=-1)

        @pl.when(i >= _NBLK - _NCACHE)
        def _():
            j = i - (_NBLK - _NCACHE)
            d = cache_ref[:, pl.ds(j * _TBLK, _TBLK), :] - mu
            dev_ref[...] = jnp.sum(d * d, axis=-1)


def _stats(E):
    def e_map(p, i):
        late = jnp.where(i < _NBLK - _NCACHE, i + _NCACHE, _NBLK - 1)
        return (0, jnp.where(p == 0, i, late), 0)

    def dev_map(p, i):
        ph1 = jnp.where(i < _NBLK - _NCACHE, i + _NCACHE, i - (_NBLK - _NCACHE))
        return (0, jnp.where(p == 0, 0, ph1))

    return pl.pallas_call(
        _stats_body,
        grid=(2, _NBLK),
        in_specs=[pl.BlockSpec((_B, _TBLK, _D), e_map)],
        out_specs=[
            pl.BlockSpec((_B, _D), lambda p, i: (0, 0)),
            pl.BlockSpec((_B, _TBLK), dev_map),
        ],
        out_shape=[
            jax.ShapeDtypeStruct((_B, _D), jnp.float32),
            jax.ShapeDtypeStruct((_B, _T), jnp.float32),
        ],
        scratch_shapes=[pltpu.VMEM((_B, _NCACHE * _TBLK, _D), jnp.float32)],
        compiler_params=pltpu.CompilerParams(
            vmem_limit_bytes=112 * 1024 * 1024),
    )(E)


def _merge16(ck, ci, nk, ni):
    kd, idd = plsc.sort_key_val(nk, ni, descending=True)
    ge = ck >= kd
    mk = jnp.where(ge, ck, kd)
    mi = jnp.where(ge, ci, idd)
    return tuple(plsc.sort_key_val(mk, mi))


def _sc_topk_body(dev_hbm, e_hbm, out_hbm, devv, rows, ssum, sem):
    c = lax.axis_index("c")
    s = lax.axis_index("s")
    sample = 2 * c + s // _NCHUNK

    @pl.when((s == 0) | (s == _NCHUNK))
    def _():
        pltpu.sync_copy(dev_hbm.at[pl.ds(sample * _T, _T)], devv)
        iota = lax.broadcasted_iota(jnp.int32, (16,), 0)

        def body(j, carry):
            ck, ci = carry
            v = devv[pl.ds(j * 16, 16)]
            return _merge16(ck, ci, v, iota + j * 16)

        init = (jnp.full((16,), _NEG, jnp.float32),
                jnp.zeros((16,), jnp.int32))
        _, fi = lax.fori_loop(0, _T // 16, body, init)

        pltpu.sync_copy(e_hbm.at[fi + sample * _T], rows)

        def dsum(dj, _):
            def inner(r, acc):
                return acc + rows[r, pl.ds(dj * 16, 16)]

            ssum[pl.ds(dj * 16, 16)] = lax.fori_loop(
                0, _K, inner, jnp.zeros((16,), jnp.float32))
            return 0

        lax.fori_loop(0, _D // 16, dsum, 0)
        pltpu.sync_copy(ssum, out_hbm.at[sample])


def _sc_topk(dev2_flat, e_flat):
    kern = pl.kernel(
        _sc_topk_body,
        out_type=jax.ShapeDtypeStruct((_B, _D), jnp.float32),
        mesh=plsc.VectorSubcoreMesh(core_axis_name="c", subcore_axis_name="s"),
        compiler_params=pltpu.CompilerParams(needs_layout_passes=False),
        scratch_types=[
            pltpu.VMEM((_T,), jnp.float32),
            pltpu.VMEM((_K, _D), jnp.float32),
            pltpu.VMEM((_D,), jnp.float32),
            pltpu.SemaphoreType.DMA,
        ],
    )
    return kern(dev2_flat, e_flat)


def _head_body(mu_ref, s_ref, w1_ref, b1_ref, w2_ref, b2_ref, out_ref):
    mu = mu_ref[...]
    S = s_ref[...]
    corr = (S - jnp.float32(_K) * mu) * jnp.float32(1.0 / _T)
    mu_rep = jnp.broadcast_to(mu[:, None, :], (_B, _N, _D)).reshape(_B * _N, _D)
    co_rep = jnp.broadcast_to(corr[:, None, :], (_B, _N, _D)).reshape(_B * _N, _D)
    ii = lax.broadcasted_iota(jnp.int32, (_B * _N, 1), 0)
    alpha = (ii % _N).astype(jnp.float32) * jnp.float32(1.0 / (_N - 1))
    Z = mu_rep - alpha * co_rep
    h = lax.dot_general(Z, w1_ref[...], (((1,), (1,)), ((), ())),
                        preferred_element_type=jnp.float32) + b1_ref[...]
    h = 0.5 * h * (1.0 + lax.erf(h * jnp.float32(0.7071067811865476)))
    out_ref[...] = lax.dot_general(h, w2_ref[...], (((1,), (1,)), ((), ())),
                                   preferred_element_type=jnp.float32) + b2_ref[...]


def _head(mu, S, W1, b1, W2, b2):
    return pl.pallas_call(
        _head_body,
        out_shape=jax.ShapeDtypeStruct((_B * _N, _NC), jnp.float32),
    )(mu, S, W1, b1.reshape(1, -1), W2, b2.reshape(1, -1))


def kernel(E, W1, b1, W2, b2):
    mu, dev2 = _stats(E)
    S = _sc_topk(dev2.reshape(-1), E.reshape(_B * _T, _D))
    return _head(mu, S, W1, b1, W2, b2)


# speedup vs baseline: 1.1911x; 1.0461x over previous
import functools

import jax
import jax.numpy as jnp
from jax import lax
from jax.experimental import pallas as pl
from jax.experimental.pallas import tpu as pltpu
from jax.experimental.pallas import tpu_sc as plsc

_B, _T, _D = 4, 8192, 768
_NC = 1000
_N = 4
_K = 16
_TBLK = 512
_NCHUNK = 8
_CHUNK = _T // _NCHUNK
_NEG = -3.0e38


_NBLK = _T // _TBLK
_NCACHE = 8


def _stats_body(e_ref, mu_ref, dev_ref, cache_ref):
    p = pl.program_id(0)
    i = pl.program_id(1)

    @pl.when(p == 0)
    def _():
        s = jnp.sum(e_ref[...], axis=1)

        @pl.when(i == 0)
        def _():
            mu_ref[...] = jnp.zeros_like(mu_ref)

        mu_ref[...] += s

        @pl.when(i < _NCACHE)
        def _():
            cache_ref[:, pl.ds(i * _TBLK, _TBLK), :] = e_ref[...]

        @pl.when(i == pl.num_programs(1) - 1)
        def _():
            mu_ref[...] *= jnp.float32(1.0 / _T)

    @pl.when(p == 1)
    def _():
        mu = mu_ref[...][:, None, :]

        @pl.when(i < _NBLK - _NCACHE)
        def _():
            d = e_ref[...] - mu
            dev_ref[...] = jnp.sum(d * d, axis=-1)

        @pl.when(i >= _NBLK - _NCACHE)
        def _():
            j = i - (_NBLK - _NCACHE)
            d = cache_ref[:, pl.ds(j * _TBLK, _TBLK), :] - mu
            dev_ref[...] = jnp.sum(d * d, axis=-1)


def _stats(E):
    def e_map(p, i):
        late = jnp.where(i < _NBLK - _NCACHE, i + _NCACHE, _NBLK - 1)
        return (0, jnp.where(p == 0, i, late), 0)

    def dev_map(p, i):
        ph1 = jnp.where(i < _NBLK - _NCACHE, i + _NCACHE, i - (_NBLK - _NCACHE))
        return (0, jnp.where(p == 0, 0, ph1))

    return pl.pallas_call(
        _stats_body,
        grid=(2, _NBLK),
        in_specs=[pl.BlockSpec((_B, _TBLK, _D), e_map)],
        out_specs=[
            pl.BlockSpec((_B, _D), lambda p, i: (0, 0)),
            pl.BlockSpec((_B, _TBLK), dev_map),
        ],
        out_shape=[
            jax.ShapeDtypeStruct((_B, _D), jnp.float32),
            jax.ShapeDtypeStruct((_B, _T), jnp.float32),
        ],
        scratch_shapes=[pltpu.VMEM((_B, _NCACHE * _TBLK, _D), jnp.float32)],
        compiler_params=pltpu.CompilerParams(
            vmem_limit_bytes=112 * 1024 * 1024),
    )(E)


def _merge16(ck, ci, nk, ni):
    kd, idd = plsc.sort_key_val(nk, ni, descending=True)
    ge = ck >= kd
    mk = jnp.where(ge, ck, kd)
    mi = jnp.where(ge, ci, idd)
    return tuple(plsc.sort_key_val(mk, mi))


def _sc_topk_body(dev_hbm, e_hbm, out_hbm, devv, rows, ssum, sem):
    c = lax.axis_index("c")
    s = lax.axis_index("s")
    sample = 2 * c + s // _NCHUNK

    @pl.when((s == 0) | (s == _NCHUNK))
    def _():
        pltpu.sync_copy(dev_hbm.at[pl.ds(sample * _T, _T)], devv)
        iota = lax.broadcasted_iota(jnp.int32, (16,), 0)

        def body(j, carry):
            out = []
            for q in range(4):
                ck, ci = carry[2 * q], carry[2 * q + 1]
                v = devv[pl.ds(j * 64 + q * 16, 16)]
                nk, ni = _merge16(ck, ci, v, iota + (j * 64 + q * 16))
                out += [nk, ni]
            return tuple(out)

        init = (jnp.full((16,), _NEG, jnp.float32),
                jnp.zeros((16,), jnp.int32)) * 4
        r = lax.fori_loop(0, _T // 64, body, init)
        ka, ia = _merge16(r[0], r[1], r[2], r[3])
        kb, ib = _merge16(r[4], r[5], r[6], r[7])
        _, fi = _merge16(ka, ia, kb, ib)

        pltpu.sync_copy(e_hbm.at[fi + sample * _T], rows)

        def dsum(dj, _):
            def inner(r, acc):
                return acc + rows[r, pl.ds(dj * 16, 16)]

            ssum[pl.ds(dj * 16, 16)] = lax.fori_loop(
                0, _K, inner, jnp.zeros((16,), jnp.float32))
            return 0

        lax.fori_loop(0, _D // 16, dsum, 0)
        pltpu.sync_copy(ssum, out_hbm.at[sample])


def _sc_topk(dev2_flat, e_flat):
    kern = pl.kernel(
        _sc_topk_body,
        out_type=jax.ShapeDtypeStruct((_B, _D), jnp.float32),
        mesh=plsc.VectorSubcoreMesh(core_axis_name="c", subcore_axis_name="s"),
        compiler_params=pltpu.CompilerParams(needs_layout_passes=False),
        scratch_types=[
            pltpu.VMEM((_T,), jnp.float32),
            pltpu.VMEM((_K, _D), jnp.float32),
            pltpu.VMEM((_D,), jnp.float32),
            pltpu.SemaphoreType.DMA,
        ],
    )
    return kern(dev2_flat, e_flat)


def _head_body(mu_ref, s_ref, w1_ref, b1_ref, w2_ref, b2_ref, out_ref):
    mu = mu_ref[...]
    S = s_ref[...]
    corr = (S - jnp.float32(_K) * mu) * jnp.float32(1.0 / _T)
    mu_rep = jnp.broadcast_to(mu[:, None, :], (_B, _N, _D)).reshape(_B * _N, _D)
    co_rep = jnp.broadcast_to(corr[:, None, :], (_B, _N, _D)).reshape(_B * _N, _D)
    ii = lax.broadcasted_iota(jnp.int32, (_B * _N, 1), 0)
    alpha = (ii % _N).astype(jnp.float32) * jnp.float32(1.0 / (_N - 1))
    Z = mu_rep - alpha * co_rep
    h = lax.dot_general(Z, w1_ref[...], (((1,), (1,)), ((), ())),
                        preferred_element_type=jnp.float32) + b1_ref[...]
    h = 0.5 * h * (1.0 + lax.erf(h * jnp.float32(0.7071067811865476)))
    out_ref[...] = lax.dot_general(h, w2_ref[...], (((1,), (1,)), ((), ())),
                                   preferred_element_type=jnp.float32) + b2_ref[...]


def _head(mu, S, W1, b1, W2, b2):
    return pl.pallas_call(
        _head_body,
        out_shape=jax.ShapeDtypeStruct((_B * _N, _NC), jnp.float32),
    )(mu, S, W1, b1.reshape(1, -1), W2, b2.reshape(1, -1))


def kernel(E, W1, b1, W2, b2):
    mu, dev2 = _stats(E)
    S = _sc_topk(dev2.reshape(-1), E.reshape(_B * _T, _D))
    return _head(mu, S, W1, b1, W2, b2)


# speedup vs baseline: 1.1998x; 1.0074x over previous
import functools

import jax
import jax.numpy as jnp
from jax import lax
from jax.experimental import pallas as pl
from jax.experimental.pallas import tpu as pltpu
from jax.experimental.pallas import tpu_sc as plsc

_B, _T, _D = 4, 8192, 768
_NC = 1000
_N = 4
_K = 16
_TBLK = 512
_NCHUNK = 8
_CHUNK = _T // _NCHUNK
_NEG = -3.0e38


_NBLK = _T // _TBLK
_NCACHE = 8


def _stats_body(e_ref, mu_ref, dev_ref, cache_ref):
    p = pl.program_id(0)
    i = pl.program_id(1)

    @pl.when(p == 0)
    def _():
        s = jnp.sum(e_ref[...], axis=1)

        @pl.when(i == 0)
        def _():
            mu_ref[...] = jnp.zeros_like(mu_ref)

        mu_ref[...] += s

        @pl.when(i < _NCACHE)
        def _():
            cache_ref[:, pl.ds(i * _TBLK, _TBLK), :] = e_ref[...]

        @pl.when(i == pl.num_programs(1) - 1)
        def _():
            mu_ref[...] *= jnp.float32(1.0 / _T)

    @pl.when(p == 1)
    def _():
        mu = mu_ref[...][:, None, :]

        @pl.when(i < _NBLK - _NCACHE)
        def _():
            d = e_ref[...] - mu
            dev_ref[...] = jnp.sum(d * d, axis=-1)

        @pl.when(i >= _NBLK - _NCACHE)
        def _():
            j = i - (_NBLK - _NCACHE)
            d = cache_ref[:, pl.ds(j * _TBLK, _TBLK), :] - mu
            dev_ref[...] = jnp.sum(d * d, axis=-1)


def _stats(E):
    def e_map(p, i):
        late = jnp.where(i < _NBLK - _NCACHE, i + _NCACHE, _NBLK - 1)
        return (0, jnp.where(p == 0, i, late), 0)

    def dev_map(p, i):
        ph1 = jnp.where(i < _NBLK - _NCACHE, i + _NCACHE, i - (_NBLK - _NCACHE))
        return (0, jnp.where(p == 0, 0, ph1))

    return pl.pallas_call(
        _stats_body,
        grid=(2, _NBLK),
        in_specs=[pl.BlockSpec((_B, _TBLK, _D), e_map)],
        out_specs=[
            pl.BlockSpec((_B, _D), lambda p, i: (0, 0)),
            pl.BlockSpec((_B, _TBLK), dev_map),
        ],
        out_shape=[
            jax.ShapeDtypeStruct((_B, _D), jnp.float32),
            jax.ShapeDtypeStruct((_B, _T), jnp.float32),
        ],
        scratch_shapes=[pltpu.VMEM((_B, _NCACHE * _TBLK, _D), jnp.float32)],
        compiler_params=pltpu.CompilerParams(
            vmem_limit_bytes=112 * 1024 * 1024),
    )(E)


def _merge16(ck, ci, nk, ni):
    kd, idd = plsc.sort_key_val(nk, ni, descending=True)
    ge = ck >= kd
    mk = jnp.where(ge, ck, kd)
    mi = jnp.where(ge, ci, idd)
    return tuple(plsc.sort_key_val(mk, mi))


def _sc_topk_body(dev_hbm, e_hbm, out_hbm, devv, rows, ssum, sem):
    c = lax.axis_index("c")
    s = lax.axis_index("s")
    sample = 2 * c + s // _NCHUNK

    @pl.when((s == 0) | (s == _NCHUNK))
    def _():
        pltpu.sync_copy(dev_hbm.at[pl.ds(sample * _T, _T)], devv)
        iota = lax.broadcasted_iota(jnp.int32, (16,), 0)

        def body(j, carry):
            out = []
            for q in range(8):
                ck, ci = carry[2 * q], carry[2 * q + 1]
                v = devv[pl.ds(j * 128 + q * 16, 16)]
                nk, ni = _merge16(ck, ci, v, iota + (j * 128 + q * 16))
                out += [nk, ni]
            return tuple(out)

        init = (jnp.full((16,), _NEG, jnp.float32),
                jnp.zeros((16,), jnp.int32)) * 8
        r = lax.fori_loop(0, _T // 128, body, init)
        m = [_merge16(r[4 * q], r[4 * q + 1], r[4 * q + 2], r[4 * q + 3])
             for q in range(4)]
        ka, ia = _merge16(m[0][0], m[0][1], m[1][0], m[1][1])
        kb, ib = _merge16(m[2][0], m[2][1], m[3][0], m[3][1])
        _, fi = _merge16(ka, ia, kb, ib)

        pltpu.sync_copy(e_hbm.at[fi + sample * _T], rows)

        def dsum(dj, _):
            def inner(r, acc):
                return acc + rows[r, pl.ds(dj * 16, 16)]

            ssum[pl.ds(dj * 16, 16)] = lax.fori_loop(
                0, _K, inner, jnp.zeros((16,), jnp.float32))
            return 0

        lax.fori_loop(0, _D // 16, dsum, 0)
        pltpu.sync_copy(ssum, out_hbm.at[sample])


def _sc_topk(dev2_flat, e_flat):
    kern = pl.kernel(
        _sc_topk_body,
        out_type=jax.ShapeDtypeStruct((_B, _D), jnp.float32),
        mesh=plsc.VectorSubcoreMesh(core_axis_name="c", subcore_axis_name="s"),
        compiler_params=pltpu.CompilerParams(needs_layout_passes=False),
        scratch_types=[
            pltpu.VMEM((_T,), jnp.float32),
            pltpu.VMEM((_K, _D), jnp.float32),
            pltpu.VMEM((_D,), jnp.float32),
            pltpu.SemaphoreType.DMA,
        ],
    )
    return kern(dev2_flat, e_flat)


def _head_body(mu_ref, s_ref, w1_ref, b1_ref, w2_ref, b2_ref, out_ref):
    mu = mu_ref[...]
    S = s_ref[...]
    corr = (S - jnp.float32(_K) * mu) * jnp.float32(1.0 / _T)
    mu_rep = jnp.broadcast_to(mu[:, None, :], (_B, _N, _D)).reshape(_B * _N, _D)
    co_rep = jnp.broadcast_to(corr[:, None, :], (_B, _N, _D)).reshape(_B * _N, _D)
    ii = lax.broadcasted_iota(jnp.int32, (_B * _N, 1), 0)
    alpha = (ii % _N).astype(jnp.float32) * jnp.float32(1.0 / (_N - 1))
    Z = mu_rep - alpha * co_rep
    h = lax.dot_general(Z, w1_ref[...], (((1,), (1,)), ((), ())),
                        preferred_element_type=jnp.float32) + b1_ref[...]
    h = 0.5 * h * (1.0 + lax.erf(h * jnp.float32(0.7071067811865476)))
    out_ref[...] = lax.dot_general(h, w2_ref[...], (((1,), (1,)), ((), ())),
                                   preferred_element_type=jnp.float32) + b2_ref[...]


def _head(mu, S, W1, b1, W2, b2):
    return pl.pallas_call(
        _head_body,
        out_shape=jax.ShapeDtypeStruct((_B * _N, _NC), jnp.float32),
    )(mu, S, W1, b1.reshape(1, -1), W2, b2.reshape(1, -1))


def kernel(E, W1, b1, W2, b2):
    mu, dev2 = _stats(E)
    S = _sc_topk(dev2.reshape(-1), E.reshape(_B * _T, _D))
    return _head(mu, S, W1, b1, W2, b2)


# speedup vs baseline: 1.2013x; 1.0013x over previous
import jax
import jax.numpy as jnp
from jax import lax
from jax.experimental import pallas as pl
from jax.experimental.pallas import tpu as pltpu
from jax.experimental.pallas import tpu_sc as plsc

_B, _T, _D = 4, 8192, 768
_NC = 1000
_N = 4
_K = 16
_TBLK = 512
_NCHUNK = 8
_NEG = -3.0e38


_NBLK = _T // _TBLK
_NCACHE = 8


def _stats_body(e_ref, mu_ref, dev_ref, cache_ref):
    p = pl.program_id(0)
    i = pl.program_id(1)

    @pl.when(p == 0)
    def _():
        s = jnp.sum(e_ref[...], axis=1)

        @pl.when(i == 0)
        def _():
            mu_ref[...] = jnp.zeros_like(mu_ref)

        mu_ref[...] += s

        @pl.when(i < _NCACHE)
        def _():
            cache_ref[:, pl.ds(i * _TBLK, _TBLK), :] = e_ref[...]

        @pl.when(i == pl.num_programs(1) - 1)
        def _():
            mu_ref[...] *= jnp.float32(1.0 / _T)

    @pl.when(p == 1)
    def _():
        mu = mu_ref[...][:, None, :]

        @pl.when(i < _NBLK - _NCACHE)
        def _():
            d = e_ref[...] - mu
            dev_ref[...] = jnp.sum(d * d, axis=-1)

        @pl.when(i >= _NBLK - _NCACHE)
        def _():
            j = i - (_NBLK - _NCACHE)
            d = cache_ref[:, pl.ds(j * _TBLK, _TBLK), :] - mu
            dev_ref[...] = jnp.sum(d * d, axis=-1)


def _stats(E):
    def e_map(p, i):
        late = jnp.where(i < _NBLK - _NCACHE, i + _NCACHE, _NBLK - 1)
        return (0, jnp.where(p == 0, i, late), 0)

    def dev_map(p, i):
        ph1 = jnp.where(i < _NBLK - _NCACHE, i + _NCACHE, i - (_NBLK - _NCACHE))
        return (0, jnp.where(p == 0, 0, ph1))

    return pl.pallas_call(
        _stats_body,
        grid=(2, _NBLK),
        in_specs=[pl.BlockSpec((_B, _TBLK, _D), e_map)],
        out_specs=[
            pl.BlockSpec((_B, _D), lambda p, i: (0, 0)),
            pl.BlockSpec((_B, _TBLK), dev_map),
        ],
        out_shape=[
            jax.ShapeDtypeStruct((_B, _D), jnp.float32),
            jax.ShapeDtypeStruct((_B, _T), jnp.float32),
        ],
        scratch_shapes=[pltpu.VMEM((_B, _NCACHE * _TBLK, _D), jnp.float32)],
        compiler_params=pltpu.CompilerParams(
            vmem_limit_bytes=112 * 1024 * 1024),
    )(E)


def _merge16(ck, ci, nk, ni):
    kd, idd = plsc.sort_key_val(nk, ni, descending=True)
    ge = ck >= kd
    mk = jnp.where(ge, ck, kd)
    mi = jnp.where(ge, ci, idd)
    return tuple(plsc.sort_key_val(mk, mi))


def _sc_topk_body(dev_hbm, e_hbm, out_hbm, devv, rows, ssum, sem):
    c = lax.axis_index("c")
    s = lax.axis_index("s")
    sample = 2 * c + s // _NCHUNK

    @pl.when((s == 0) | (s == _NCHUNK))
    def _():
        pltpu.sync_copy(dev_hbm.at[pl.ds(sample * _T, _T)], devv)
        iota = lax.broadcasted_iota(jnp.int32, (16,), 0)

        def body(j, carry):
            out = []
            for q in range(8):
                ck, ci = carry[2 * q], carry[2 * q + 1]
                v = devv[pl.ds(j * 128 + q * 16, 16)]
                nk, ni = _merge16(ck, ci, v, iota + (j * 128 + q * 16))
                out += [nk, ni]
            return tuple(out)

        init = (jnp.full((16,), _NEG, jnp.float32),
                jnp.zeros((16,), jnp.int32)) * 8
        r = lax.fori_loop(0, _T // 128, body, init)
        m = [_merge16(r[4 * q], r[4 * q + 1], r[4 * q + 2], r[4 * q + 3])
             for q in range(4)]
        ka, ia = _merge16(m[0][0], m[0][1], m[1][0], m[1][1])
        kb, ib = _merge16(m[2][0], m[2][1], m[3][0], m[3][1])
        _, fi = _merge16(ka, ia, kb, ib)

        pltpu.sync_copy(e_hbm.at[fi + sample * _T], rows)

        def dsum(dj, _):
            def inner(r, acc):
                return acc + rows[r, pl.ds(dj * 16, 16)]

            ssum[pl.ds(dj * 16, 16)] = lax.fori_loop(
                0, _K, inner, jnp.zeros((16,), jnp.float32))
            return 0

        lax.fori_loop(0, _D // 16, dsum, 0)
        pltpu.sync_copy(ssum, out_hbm.at[sample])


def _sc_topk(dev2_flat, e_flat):
    kern = pl.kernel(
        _sc_topk_body,
        out_type=jax.ShapeDtypeStruct((_B, _D), jnp.float32),
        mesh=plsc.VectorSubcoreMesh(core_axis_name="c", subcore_axis_name="s"),
        compiler_params=pltpu.CompilerParams(needs_layout_passes=False),
        scratch_types=[
            pltpu.VMEM((_T,), jnp.float32),
            pltpu.VMEM((_K, _D), jnp.float32),
            pltpu.VMEM((_D,), jnp.float32),
            pltpu.SemaphoreType.DMA,
        ],
    )
    return kern(dev2_flat, e_flat)


def _head_body(mu_ref, s_ref, w1_ref, b1_ref, w2_ref, b2_ref, out_ref):
    mu = mu_ref[...]
    S = s_ref[...]
    corr = (S - jnp.float32(_K) * mu) * jnp.float32(1.0 / _T)
    mu_rep = jnp.broadcast_to(mu[:, None, :], (_B, _N, _D)).reshape(_B * _N, _D)
    co_rep = jnp.broadcast_to(corr[:, None, :], (_B, _N, _D)).reshape(_B * _N, _D)
    ii = lax.broadcasted_iota(jnp.int32, (_B * _N, 1), 0)
    alpha = (ii % _N).astype(jnp.float32) * jnp.float32(1.0 / (_N - 1))
    Z = mu_rep - alpha * co_rep
    h = lax.dot_general(Z, w1_ref[...], (((1,), (1,)), ((), ())),
                        preferred_element_type=jnp.float32) + b1_ref[...]
    h = 0.5 * h * (1.0 + lax.erf(h * jnp.float32(0.7071067811865476)))
    out_ref[...] = lax.dot_general(h, w2_ref[...], (((1,), (1,)), ((), ())),
                                   preferred_element_type=jnp.float32) + b2_ref[...]


def _head(mu, S, W1, b1, W2, b2):
    return pl.pallas_call(
        _head_body,
        out_shape=jax.ShapeDtypeStruct((_B * _N, _NC), jnp.float32),
    )(mu, S, W1, b1.reshape(1, -1), W2, b2.reshape(1, -1))


def kernel(E, W1, b1, W2, b2):
    mu, dev2 = _stats(E)
    S = _sc_topk(dev2.reshape(-1), E.reshape(_B * _T, _D))
    return _head(mu, S, W1, b1, W2, b2)
